# Initial kernel scaffold; baseline (speedup 1.0000x reference)
import jax
import jax.numpy as jnp
from jax.experimental import pallas as pl


def kernel(hidden_states, pooled_output, Wp1, bp1, Wp2, bp2, W1l, b1l, W1r, b1r, att1, bias1, W2l, b2l, W2r, b2r, att2, bias2, Wlin, blin):
    raise NotImplementedError("write your pallas kernel here")


# speedup vs baseline: 73.6986x; 73.6986x over previous
import jax
import jax.numpy as jnp
from jax.experimental import pallas as pl
from jax.experimental.pallas import tpu as pltpu

_BS = 8
_S = 4096
_DIN = 768
_DH = 512
_H1, _C1 = 4, 128
_N_NODE = _S + 1

_CH = 1024
_NCH = (_BS * _S) // _CH


def _leaky(x):
    return jnp.where(x >= 0, x, 0.2 * x)


def _gelu(x):
    return 0.5 * x * (1.0 + jax.lax.erf(x * 0.7071067811865476))


def _proj_kernel(hs_ref, wp1_ref, bp1_ref, wp2_ref, bp2_ref, out_ref):
    h = hs_ref[...].astype(jnp.bfloat16)
    t = jnp.dot(h, wp1_ref[...], preferred_element_type=jnp.float32) + bp1_ref[...]
    t = _gelu(t).astype(jnp.bfloat16)
    p = jnp.dot(t, wp2_ref[...], preferred_element_type=jnp.float32) + bp2_ref[...]
    p = _gelu(p)
    out_ref[...] = p.astype(jnp.bfloat16)


def _gat_kernel(p_ref, pool_ref, w1l_ref, b1l_ref, w1r_ref, b1r_ref, att1_ref,
                bias1_ref, w2l_ref, b2l_ref, w2r_ref, b2r_ref, att2_ref,
                bias2_ref, wlin_ref, blin_ref, g_ref, mean_ref):
    P = p_ref[0]
    pool = pool_ref[0]
    poolb = pool.astype(jnp.bfloat16)

    Ll = jnp.dot(P, w1l_ref[...], preferred_element_type=jnp.float32) + b1l_ref[...]
    Lr = jnp.dot(P, w1r_ref[...], preferred_element_type=jnp.float32) + b1r_ref[...]
    hl = jnp.dot(poolb, w1l_ref[...], preferred_element_type=jnp.float32) + b1l_ref[...]
    hr = jnp.dot(poolb, w1r_ref[...], preferred_element_type=jnp.float32) + b1r_ref[...]
    att1 = att1_ref[...]
    y_parts, y0_parts = [], []
    for h in range(_H1):
        sl = slice(h * _C1, (h + 1) * _C1)
        Llh, Lrh, hlh, hrh = Ll[:, sl], Lr[:, sl], hl[:, sl], hr[:, sl]
        ath = att1[h:h + 1, :]
        lg_s = jnp.sum(_leaky(Llh + Lrh) * ath, axis=1, keepdims=True)
        lg_h = jnp.sum(_leaky(hlh + Lrh) * ath, axis=1, keepdims=True)
        m = jnp.maximum(lg_s, lg_h)
        es = jnp.exp(lg_s - m)
        eh = jnp.exp(lg_h - m)
        den = es + eh + 1e-16
        y_parts.append((eh * hlh + es * Llh) / den)
        lg_b = jnp.sum(_leaky(Llh + hrh) * ath, axis=1, keepdims=True)
        lg_0 = jnp.sum(_leaky(hlh + hrh) * ath, axis=1, keepdims=True)
        m0 = jnp.maximum(jnp.max(lg_b, axis=0, keepdims=True), lg_0)
        eb = jnp.exp(lg_b - m0)
        e0 = jnp.exp(lg_0 - m0)
        den0 = jnp.sum(eb, axis=0, keepdims=True) + e0 + 1e-16
        y0_parts.append(
            (jnp.sum(eb * Llh, axis=0, keepdims=True) + e0 * hlh) / den0)
    Y = jnp.concatenate(y_parts, axis=1) + bias1_ref[...]
    y0 = jnp.concatenate(y0_parts, axis=1) + bias1_ref[...]
    Y = _gelu(Y).astype(jnp.bfloat16)
    y0 = _gelu(y0).astype(jnp.bfloat16)

    Zl = jnp.dot(Y, w2l_ref[...], preferred_element_type=jnp.float32) + b2l_ref[...]
    Zr = jnp.dot(Y, w2r_ref[...], preferred_element_type=jnp.float32) + b2r_ref[...]
    hl2 = jnp.dot(y0, w2l_ref[...], preferred_element_type=jnp.float32) + b2l_ref[...]
    hr2 = jnp.dot(y0, w2r_ref[...], preferred_element_type=jnp.float32) + b2r_ref[...]
    at2 = att2_ref[...]
    lg_s = jnp.sum(_leaky(Zl + Zr) * at2, axis=1, keepdims=True)
    lg_h = jnp.sum(_leaky(hl2 + Zr) * at2, axis=1, keepdims=True)
    m = jnp.maximum(lg_s, lg_h)
    es = jnp.exp(lg_s - m)
    eh = jnp.exp(lg_h - m)
    den = es + eh + 1e-16
    Z = (eh * hl2 + es * Zl) / den + bias2_ref[...]
    lg_b = jnp.sum(_leaky(Zl + hr2) * at2, axis=1, keepdims=True)
    lg_0 = jnp.sum(_leaky(hl2 + hr2) * at2, axis=1, keepdims=True)
    m0 = jnp.maximum(jnp.max(lg_b, axis=0, keepdims=True), lg_0)
    eb = jnp.exp(lg_b - m0)
    e0 = jnp.exp(lg_0 - m0)
    den0 = jnp.sum(eb, axis=0, keepdims=True) + e0 + 1e-16
    z0 = (jnp.sum(eb * Zl, axis=0, keepdims=True) + e0 * hl2) / den0 + bias2_ref[...]

    mean_ref[0] = (jnp.sum(Z, axis=0, keepdims=True) + z0) * (1.0 / _N_NODE)
    g = jnp.dot(z0, wlin_ref[...], preferred_element_type=jnp.float32) + blin_ref[...]
    g_ref[0] = g + pool


def kernel(hidden_states, pooled_output, Wp1, bp1, Wp2, bp2, W1l, b1l, W1r,
           b1r, att1, bias1, W2l, b2l, W2r, b2r, att2, bias2, Wlin, blin):
    f32, bf16 = jnp.float32, jnp.bfloat16
    hs = hidden_states[0].reshape(_BS * _S, _DIN)

    p = pl.pallas_call(
        _proj_kernel,
        grid=(_NCH,),
        in_specs=[
            pl.BlockSpec((_CH, _DIN), lambda i: (i, 0)),
            pl.BlockSpec((_DIN, _DH), lambda i: (0, 0)),
            pl.BlockSpec((1, _DH), lambda i: (0, 0)),
            pl.BlockSpec((_DH, _DH), lambda i: (0, 0)),
            pl.BlockSpec((1, _DH), lambda i: (0, 0)),
        ],
        out_specs=pl.BlockSpec((_CH, _DH), lambda i: (i, 0)),
        out_shape=jax.ShapeDtypeStruct((_BS * _S, _DH), bf16),
        compiler_params=pltpu.CompilerParams(
            dimension_semantics=("parallel",)),
    )(hs, Wp1.astype(bf16), bp1.reshape(1, _DH), Wp2.astype(bf16),
      bp2.reshape(1, _DH))

    g, mean = pl.pallas_call(
        _gat_kernel,
        grid=(_BS,),
        in_specs=[
            pl.BlockSpec((1, _S, _DH), lambda i: (i, 0, 0)),
            pl.BlockSpec((1, 1, _DH), lambda i: (i, 0, 0)),
            pl.BlockSpec((_DH, _DH), lambda i: (0, 0)),
            pl.BlockSpec((1, _DH), lambda i: (0, 0)),
            pl.BlockSpec((_DH, _DH), lambda i: (0, 0)),
            pl.BlockSpec((1, _DH), lambda i: (0, 0)),
            pl.BlockSpec((_H1, _C1), lambda i: (0, 0)),
            pl.BlockSpec((1, _DH), lambda i: (0, 0)),
            pl.BlockSpec((_DH, _DH), lambda i: (0, 0)),
            pl.BlockSpec((1, _DH), lambda i: (0, 0)),
            pl.BlockSpec((_DH, _DH), lambda i: (0, 0)),
            pl.BlockSpec((1, _DH), lambda i: (0, 0)),
            pl.BlockSpec((1, _DH), lambda i: (0, 0)),
            pl.BlockSpec((1, _DH), lambda i: (0, 0)),
            pl.BlockSpec((_DH, _DH), lambda i: (0, 0)),
            pl.BlockSpec((1, _DH), lambda i: (0, 0)),
        ],
        out_specs=[
            pl.BlockSpec((1, 1, _DH), lambda i: (i, 0, 0)),
            pl.BlockSpec((1, 1, _DH), lambda i: (i, 0, 0)),
        ],
        out_shape=[
            jax.ShapeDtypeStruct((_BS, 1, _DH), f32),
            jax.ShapeDtypeStruct((_BS, 1, _DH), f32),
        ],
        compiler_params=pltpu.CompilerParams(
            dimension_semantics=("parallel",)),
    )(p.reshape(_BS, _S, _DH), pooled_output.reshape(_BS, 1, _DH),
      W1l.astype(bf16), b1l.reshape(1, _DH),
      W1r.astype(bf16), b1r.reshape(1, _DH),
      att1, bias1.reshape(1, _DH),
      W2l.astype(bf16), b2l.reshape(1, _DH),
      W2r.astype(bf16), b2r.reshape(1, _DH),
      att2.reshape(1, _DH), bias2.reshape(1, _DH),
      Wlin, blin.reshape(1, _DH))
    return (g.reshape(_BS, _DH), mean.reshape(_BS, _DH))


# speedup vs baseline: 79.1069x; 1.0734x over previous
import jax
import jax.numpy as jnp
from jax.experimental import pallas as pl
from jax.experimental.pallas import tpu as pltpu

_BS = 8
_S = 4096
_DIN = 768
_DH = 512
_H1, _C1 = 4, 128
_N_NODE = _S + 1

_CH = 1024
_NCH = (_BS * _S) // _CH


def _leaky(x):
    return jnp.where(x >= 0, x, 0.2 * x)


def _gelu(x):
    return 0.5 * x * (1.0 + jax.lax.erf(x * 0.7071067811865476))


def _proj_kernel(hs_ref, wp1_ref, bp1_ref, wp2_ref, bp2_ref, out_ref):
    h = hs_ref[...].astype(jnp.bfloat16)
    t = jnp.dot(h, wp1_ref[...], preferred_element_type=jnp.float32) + bp1_ref[...]
    t = _gelu(t).astype(jnp.bfloat16)
    p = jnp.dot(t, wp2_ref[...], preferred_element_type=jnp.float32) + bp2_ref[...]
    p = _gelu(p)
    out_ref[...] = p.astype(jnp.bfloat16)


def _colsum_T(eb, X):
    return jax.lax.dot_general(
        eb, X, (((0,), (0,)), ((), ())), preferred_element_type=jnp.float32)


def _gat_kernel(p_ref, pool_ref, w1l_ref, b1l_ref, w1r_ref, b1r_ref, a1_ref,
                bias1_ref, w2l_ref, b2l_ref, w2r_ref, b2r_ref, a2_ref,
                bias2_ref, wlin_ref, blin_ref, g_ref, mean_ref):
    bf16, f32 = jnp.bfloat16, jnp.float32
    P = p_ref[0]
    pool = pool_ref[0]
    poolb = pool.astype(bf16)

    A1 = a1_ref[...]
    Ll = jnp.dot(P, w1l_ref[...], preferred_element_type=f32) + b1l_ref[...]
    Lr = jnp.dot(P, w1r_ref[...], preferred_element_type=f32) + b1r_ref[...]
    hl = jnp.dot(poolb, w1l_ref[...], preferred_element_type=f32) + b1l_ref[...]
    hr = jnp.dot(poolb, w1r_ref[...], preferred_element_type=f32) + b1r_ref[...]
    Llb = Ll.astype(bf16)
    lg_s = jnp.dot(_leaky(Ll + Lr).astype(bf16), A1, preferred_element_type=f32)
    lg_h = jnp.dot(_leaky(hl + Lr).astype(bf16), A1, preferred_element_type=f32)
    lg_b = jnp.dot(_leaky(Ll + hr).astype(bf16), A1, preferred_element_type=f32)
    lg_0 = jnp.dot(_leaky(hl + hr).astype(bf16), A1, preferred_element_type=f32)
    m0 = jnp.maximum(jnp.max(lg_b, axis=0, keepdims=True), lg_0)
    eb = jnp.exp(lg_b - m0)
    e0 = jnp.exp(lg_0 - m0)
    den0 = jnp.sum(eb, axis=0, keepdims=True) + e0 + 1e-16
    M = _colsum_T(eb.astype(bf16), Llb)
    ws = jax.nn.sigmoid(lg_s - lg_h)
    y_parts, y0_parts = [], []
    for h in range(_H1):
        sl = slice(h * _C1, (h + 1) * _C1)
        Llh, hlh = Ll[:, sl], hl[:, sl]
        wsh = ws[:, h:h + 1]
        y_parts.append(hlh + wsh * (Llh - hlh))
        y0_parts.append((M[h:h + 1, sl] + e0[:, h:h + 1] * hlh)
                        / den0[:, h:h + 1])
    Y = jnp.concatenate(y_parts, axis=1) + bias1_ref[...]
    y0 = jnp.concatenate(y0_parts, axis=1) + bias1_ref[...]
    Y = _gelu(Y).astype(bf16)
    y0 = _gelu(y0).astype(bf16)

    A2 = a2_ref[...]
    Zl = jnp.dot(Y, w2l_ref[...], preferred_element_type=f32) + b2l_ref[...]
    Zr = jnp.dot(Y, w2r_ref[...], preferred_element_type=f32) + b2r_ref[...]
    hl2 = jnp.dot(y0, w2l_ref[...], preferred_element_type=f32) + b2l_ref[...]
    hr2 = jnp.dot(y0, w2r_ref[...], preferred_element_type=f32) + b2r_ref[...]
    Zlb = Zl.astype(bf16)
    lg_s = jnp.dot(_leaky(Zl + Zr).astype(bf16), A2, preferred_element_type=f32)
    lg_h = jnp.dot(_leaky(hl2 + Zr).astype(bf16), A2, preferred_element_type=f32)
    lg_b = jnp.dot(_leaky(Zl + hr2).astype(bf16), A2, preferred_element_type=f32)
    lg_0 = jnp.dot(_leaky(hl2 + hr2).astype(bf16), A2, preferred_element_type=f32)
    ws = jax.nn.sigmoid(lg_s - lg_h)
    Z = hl2 + ws * (Zl - hl2) + bias2_ref[...]
    m0 = jnp.maximum(jnp.max(lg_b, axis=0, keepdims=True), lg_0)
    eb = jnp.exp(lg_b - m0)
    e0 = jnp.exp(lg_0 - m0)
    den0 = jnp.sum(eb, axis=0, keepdims=True) + e0 + 1e-16
    M2 = _colsum_T(eb.astype(bf16), Zlb)
    z0 = (M2 + e0 * hl2) / den0 + bias2_ref[...]

    mean_ref[0] = (jnp.sum(Z, axis=0, keepdims=True) + z0) * (1.0 / _N_NODE)
    g = jnp.dot(z0, wlin_ref[...], preferred_element_type=f32) + blin_ref[...]
    g_ref[0] = g + pool


def kernel(hidden_states, pooled_output, Wp1, bp1, Wp2, bp2, W1l, b1l, W1r,
           b1r, att1, bias1, W2l, b2l, W2r, b2r, att2, bias2, Wlin, blin):
    f32, bf16 = jnp.float32, jnp.bfloat16
    hs = hidden_states[0].reshape(_BS * _S, _DIN)

    p = pl.pallas_call(
        _proj_kernel,
        grid=(_NCH,),
        in_specs=[
            pl.BlockSpec((_CH, _DIN), lambda i: (i, 0)),
            pl.BlockSpec((_DIN, _DH), lambda i: (0, 0)),
            pl.BlockSpec((1, _DH), lambda i: (0, 0)),
            pl.BlockSpec((_DH, _DH), lambda i: (0, 0)),
            pl.BlockSpec((1, _DH), lambda i: (0, 0)),
        ],
        out_specs=pl.BlockSpec((_CH, _DH), lambda i: (i, 0)),
        out_shape=jax.ShapeDtypeStruct((_BS * _S, _DH), bf16),
        compiler_params=pltpu.CompilerParams(
            dimension_semantics=("parallel",)),
    )(hs, Wp1.astype(bf16), bp1.reshape(1, _DH), Wp2.astype(bf16),
      bp2.reshape(1, _DH))

    g_mean = pl.pallas_call(
        _gat_kernel,
        grid=(_BS,),
        in_specs=[
            pl.BlockSpec((1, _S, _DH), lambda i: (i, 0, 0)),
            pl.BlockSpec((1, 1, _DH), lambda i: (i, 0, 0)),
            pl.BlockSpec((_DH, _DH), lambda i: (0, 0)),
            pl.BlockSpec((1, _DH), lambda i: (0, 0)),
            pl.BlockSpec((_DH, _DH), lambda i: (0, 0)),
            pl.BlockSpec((1, _DH), lambda i: (0, 0)),
            pl.BlockSpec((_DH, _H1), lambda i: (0, 0)),
            pl.BlockSpec((1, _DH), lambda i: (0, 0)),
            pl.BlockSpec((_DH, _DH), lambda i: (0, 0)),
            pl.BlockSpec((1, _DH), lambda i: (0, 0)),
            pl.BlockSpec((_DH, _DH), lambda i: (0, 0)),
            pl.BlockSpec((1, _DH), lambda i: (0, 0)),
            pl.BlockSpec((_DH, 1), lambda i: (0, 0)),
            pl.BlockSpec((1, _DH), lambda i: (0, 0)),
            pl.BlockSpec((_DH, _DH), lambda i: (0, 0)),
            pl.BlockSpec((1, _DH), lambda i: (0, 0)),
        ],
        out_specs=[
            pl.BlockSpec((1, 1, _DH), lambda i: (i, 0, 0)),
            pl.BlockSpec((1, 1, _DH), lambda i: (i, 0, 0)),
        ],
        out_shape=[
            jax.ShapeDtypeStruct((_BS, 1, _DH), f32),
            jax.ShapeDtypeStruct((_BS, 1, _DH), f32),
        ],
        compiler_params=pltpu.CompilerParams(
            dimension_semantics=("parallel",)),
    )
    a1 = jnp.zeros((_DH, _H1), f32)
    for h in range(_H1):
        a1 = a1.at[h * _C1:(h + 1) * _C1, h].set(att1[h])
    g, mean = g_mean(
        p.reshape(_BS, _S, _DH), pooled_output.reshape(_BS, 1, _DH),
        W1l.astype(bf16), b1l.reshape(1, _DH),
        W1r.astype(bf16), b1r.reshape(1, _DH),
        a1.astype(bf16), bias1.reshape(1, _DH),
        W2l.astype(bf16), b2l.reshape(1, _DH),
        W2r.astype(bf16), b2r.reshape(1, _DH),
        att2.reshape(_DH, 1).astype(bf16), bias2.reshape(1, _DH),
        Wlin, blin.reshape(1, _DH))
    return (g.reshape(_BS, _DH), mean.reshape(_BS, _DH))


# speedup vs baseline: 84.5843x; 1.0692x over previous
import jax
import jax.numpy as jnp
from jax.experimental import pallas as pl
from jax.experimental.pallas import tpu as pltpu

_BS = 8
_S = 4096
_DIN = 768
_DH = 512
_H1, _C1 = 4, 128
_N_NODE = _S + 1

_CH = 1024
_NCH = (_BS * _S) // _CH


def _leaky(x):
    return jnp.maximum(x, x * jnp.asarray(0.2, x.dtype))


def _gelu(x):
    return 0.5 * x * (1.0 + jax.lax.erf(x * 0.7071067811865476))


def _proj_kernel(hs_ref, wp1_ref, bp1_ref, wp2_ref, bp2_ref, out_ref):
    h = hs_ref[...].astype(jnp.bfloat16)
    t = jnp.dot(h, wp1_ref[...], preferred_element_type=jnp.float32) + bp1_ref[...]
    t = _gelu(t).astype(jnp.bfloat16)
    p = jnp.dot(t, wp2_ref[...], preferred_element_type=jnp.float32) + bp2_ref[...]
    p = _gelu(p)
    out_ref[...] = p.astype(jnp.bfloat16)


def _colsum_T(eb, X):
    return jax.lax.dot_general(
        eb, X, (((0,), (0,)), ((), ())), preferred_element_type=jnp.float32)


def _gat_kernel(p_ref, pool_ref, w1l_ref, b1l_ref, w1r_ref, b1r_ref, a1_ref,
                bias1_ref, w2l_ref, b2l_ref, w2r_ref, b2r_ref, a2_ref,
                bias2_ref, wlin_ref, blin_ref, g_ref, mean_ref):
    bf16, f32 = jnp.bfloat16, jnp.float32
    P = p_ref[0]
    pool = pool_ref[0]
    poolb = pool.astype(bf16)

    A1 = a1_ref[...]
    b1lb = b1l_ref[...].astype(bf16)
    b1rb = b1r_ref[...].astype(bf16)
    Llb = jnp.dot(P, w1l_ref[...], preferred_element_type=f32).astype(bf16) + b1lb
    Lrb = jnp.dot(P, w1r_ref[...], preferred_element_type=f32).astype(bf16) + b1rb
    hl = jnp.dot(poolb, w1l_ref[...], preferred_element_type=f32) + b1l_ref[...]
    hr = jnp.dot(poolb, w1r_ref[...], preferred_element_type=f32) + b1r_ref[...]
    hlb = hl.astype(bf16)
    hrb = hr.astype(bf16)
    lg_s = jnp.dot(_leaky(Llb + Lrb), A1, preferred_element_type=f32)
    lg_h = jnp.dot(_leaky(hlb + Lrb), A1, preferred_element_type=f32)
    lg_b = jnp.dot(_leaky(Llb + hrb), A1, preferred_element_type=f32)
    lg_0 = jnp.dot(_leaky(hlb + hrb), A1, preferred_element_type=f32)
    m0 = jnp.maximum(jnp.max(lg_b, axis=0, keepdims=True), lg_0)
    eb = jnp.exp(lg_b - m0)
    e0 = jnp.exp(lg_0 - m0)
    den0 = jnp.sum(eb, axis=0, keepdims=True) + e0 + 1e-16
    M = _colsum_T(eb.astype(bf16), Llb)
    ws = jax.nn.sigmoid(lg_s - lg_h).astype(bf16)
    y_parts, y0_parts = [], []
    for h in range(_H1):
        sl = slice(h * _C1, (h + 1) * _C1)
        Llh, hlh = Llb[:, sl], hlb[:, sl]
        wsh = ws[:, h:h + 1]
        y_parts.append(hlh + wsh * (Llh - hlh))
        y0_parts.append((M[h:h + 1, sl] + e0[:, h:h + 1] * hl[:, sl])
                        / den0[:, h:h + 1])
    Yb = jnp.concatenate(y_parts, axis=1)
    y0 = jnp.concatenate(y0_parts, axis=1) + bias1_ref[...]
    Y = _gelu(Yb.astype(f32) + bias1_ref[...]).astype(bf16)
    y0 = _gelu(y0).astype(bf16)

    A2 = a2_ref[...]
    b2lb = b2l_ref[...].astype(bf16)
    b2rb = b2r_ref[...].astype(bf16)
    Zlb = jnp.dot(Y, w2l_ref[...], preferred_element_type=f32).astype(bf16) + b2lb
    Zrb = jnp.dot(Y, w2r_ref[...], preferred_element_type=f32).astype(bf16) + b2rb
    hl2 = jnp.dot(y0, w2l_ref[...], preferred_element_type=f32) + b2l_ref[...]
    hr2 = jnp.dot(y0, w2r_ref[...], preferred_element_type=f32) + b2r_ref[...]
    hl2b = hl2.astype(bf16)
    hr2b = hr2.astype(bf16)
    lg_s = jnp.dot(_leaky(Zlb + Zrb), A2, preferred_element_type=f32)
    lg_h = jnp.dot(_leaky(hl2b + Zrb), A2, preferred_element_type=f32)
    lg_b = jnp.dot(_leaky(Zlb + hr2b), A2, preferred_element_type=f32)
    lg_0 = jnp.dot(_leaky(hl2b + hr2b), A2, preferred_element_type=f32)
    ws = jax.nn.sigmoid(lg_s - lg_h)
    Z = hl2 + ws * (Zlb.astype(f32) - hl2) + bias2_ref[...]
    m0 = jnp.maximum(jnp.max(lg_b, axis=0, keepdims=True), lg_0)
    eb = jnp.exp(lg_b - m0)
    e0 = jnp.exp(lg_0 - m0)
    den0 = jnp.sum(eb, axis=0, keepdims=True) + e0 + 1e-16
    M2 = _colsum_T(eb.astype(bf16), Zlb)
    z0 = (M2 + e0 * hl2) / den0 + bias2_ref[...]

    mean_ref[0] = (jnp.sum(Z, axis=0, keepdims=True) + z0) * (1.0 / _N_NODE)
    g = jnp.dot(z0, wlin_ref[...], preferred_element_type=f32) + blin_ref[...]
    g_ref[0] = g + pool


def kernel(hidden_states, pooled_output, Wp1, bp1, Wp2, bp2, W1l, b1l, W1r,
           b1r, att1, bias1, W2l, b2l, W2r, b2r, att2, bias2, Wlin, blin):
    f32, bf16 = jnp.float32, jnp.bfloat16
    hs = hidden_states[0].reshape(_BS * _S, _DIN)

    p = pl.pallas_call(
        _proj_kernel,
        grid=(_NCH,),
        in_specs=[
            pl.BlockSpec((_CH, _DIN), lambda i: (i, 0)),
            pl.BlockSpec((_DIN, _DH), lambda i: (0, 0)),
            pl.BlockSpec((1, _DH), lambda i: (0, 0)),
            pl.BlockSpec((_DH, _DH), lambda i: (0, 0)),
            pl.BlockSpec((1, _DH), lambda i: (0, 0)),
        ],
        out_specs=pl.BlockSpec((_CH, _DH), lambda i: (i, 0)),
        out_shape=jax.ShapeDtypeStruct((_BS * _S, _DH), bf16),
        compiler_params=pltpu.CompilerParams(
            dimension_semantics=("parallel",)),
    )(hs, Wp1.astype(bf16), bp1.reshape(1, _DH), Wp2.astype(bf16),
      bp2.reshape(1, _DH))

    g_mean = pl.pallas_call(
        _gat_kernel,
        grid=(_BS,),
        in_specs=[
            pl.BlockSpec((1, _S, _DH), lambda i: (i, 0, 0)),
            pl.BlockSpec((1, 1, _DH), lambda i: (i, 0, 0)),
            pl.BlockSpec((_DH, _DH), lambda i: (0, 0)),
            pl.BlockSpec((1, _DH), lambda i: (0, 0)),
            pl.BlockSpec((_DH, _DH), lambda i: (0, 0)),
            pl.BlockSpec((1, _DH), lambda i: (0, 0)),
            pl.BlockSpec((_DH, _H1), lambda i: (0, 0)),
            pl.BlockSpec((1, _DH), lambda i: (0, 0)),
            pl.BlockSpec((_DH, _DH), lambda i: (0, 0)),
            pl.BlockSpec((1, _DH), lambda i: (0, 0)),
            pl.BlockSpec((_DH, _DH), lambda i: (0, 0)),
            pl.BlockSpec((1, _DH), lambda i: (0, 0)),
            pl.BlockSpec((_DH, 1), lambda i: (0, 0)),
            pl.BlockSpec((1, _DH), lambda i: (0, 0)),
            pl.BlockSpec((_DH, _DH), lambda i: (0, 0)),
            pl.BlockSpec((1, _DH), lambda i: (0, 0)),
        ],
        out_specs=[
            pl.BlockSpec((1, 1, _DH), lambda i: (i, 0, 0)),
            pl.BlockSpec((1, 1, _DH), lambda i: (i, 0, 0)),
        ],
        out_shape=[
            jax.ShapeDtypeStruct((_BS, 1, _DH), f32),
            jax.ShapeDtypeStruct((_BS, 1, _DH), f32),
        ],
        compiler_params=pltpu.CompilerParams(
            dimension_semantics=("parallel",)),
    )
    a1 = jnp.zeros((_DH, _H1), f32)
    for h in range(_H1):
        a1 = a1.at[h * _C1:(h + 1) * _C1, h].set(att1[h])
    g, mean = g_mean(
        p.reshape(_BS, _S, _DH), pooled_output.reshape(_BS, 1, _DH),
        W1l.astype(bf16), b1l.reshape(1, _DH),
        W1r.astype(bf16), b1r.reshape(1, _DH),
        a1.astype(bf16), bias1.reshape(1, _DH),
        W2l.astype(bf16), b2l.reshape(1, _DH),
        W2r.astype(bf16), b2r.reshape(1, _DH),
        att2.reshape(_DH, 1).astype(bf16), bias2.reshape(1, _DH),
        Wlin, blin.reshape(1, _DH))
    return (g.reshape(_BS, _DH), mean.reshape(_BS, _DH))


# speedup vs baseline: 87.0936x; 1.0297x over previous
import jax
import jax.numpy as jnp
from jax.experimental import pallas as pl
from jax.experimental.pallas import tpu as pltpu

_BS = 8
_S = 4096
_DIN = 768
_DH = 512
_H1, _C1 = 4, 128
_N_NODE = _S + 1

_CH = 1024
_NCH = (_BS * _S) // _CH


def _leaky(x):
    return jnp.maximum(x, x * jnp.asarray(0.2, x.dtype))


def _gelu(x):
    return 0.5 * x * (1.0 + jax.lax.erf(x * 0.7071067811865476))


def _proj_kernel(hs_ref, wp1_ref, bp1_ref, wp2_ref, bp2_ref, out_ref):
    h = hs_ref[...].astype(jnp.bfloat16)
    t = jnp.dot(h, wp1_ref[...], preferred_element_type=jnp.float32) + bp1_ref[...]
    t = _gelu(t).astype(jnp.bfloat16)
    p = jnp.dot(t, wp2_ref[...], preferred_element_type=jnp.float32) + bp2_ref[...]
    p = _gelu(p)
    out_ref[...] = p.astype(jnp.bfloat16)


def _colsum_T(eb, X):
    return jax.lax.dot_general(
        eb, X, (((0,), (0,)), ((), ())), preferred_element_type=jnp.float32)


def _fused_kernel(hs_ref, pool_ref, wp1_ref, bp1_ref, wp2_ref, bp2_ref,
                  w1l_ref, b1l_ref, w1r_ref, b1r_ref, a1_ref,
                  bias1_ref, w2l_ref, b2l_ref, w2r_ref, b2r_ref, a2_ref,
                  bias2_ref, wlin_ref, blin_ref, g_ref, mean_ref):
    bf16, f32 = jnp.bfloat16, jnp.float32
    h = hs_ref[0].astype(bf16)
    t = jnp.dot(h, wp1_ref[...], preferred_element_type=f32) + bp1_ref[...]
    t = _gelu(t).astype(bf16)
    t = jnp.dot(t, wp2_ref[...], preferred_element_type=f32) + bp2_ref[...]
    P = _gelu(t).astype(bf16)
    pool = pool_ref[0]
    poolb = pool.astype(bf16)

    A1 = a1_ref[...]
    b1lb = b1l_ref[...].astype(bf16)
    b1rb = b1r_ref[...].astype(bf16)
    Llb = jnp.dot(P, w1l_ref[...], preferred_element_type=f32).astype(bf16) + b1lb
    Lrb = jnp.dot(P, w1r_ref[...], preferred_element_type=f32).astype(bf16) + b1rb
    hl = jnp.dot(poolb, w1l_ref[...], preferred_element_type=f32) + b1l_ref[...]
    hr = jnp.dot(poolb, w1r_ref[...], preferred_element_type=f32) + b1r_ref[...]
    hlb = hl.astype(bf16)
    hrb = hr.astype(bf16)
    lg_s = jnp.dot(_leaky(Llb + Lrb), A1, preferred_element_type=f32)
    lg_h = jnp.dot(_leaky(hlb + Lrb), A1, preferred_element_type=f32)
    lg_b = jnp.dot(_leaky(Llb + hrb), A1, preferred_element_type=f32)
    lg_0 = jnp.dot(_leaky(hlb + hrb), A1, preferred_element_type=f32)
    m0 = jnp.maximum(jnp.max(lg_b, axis=0, keepdims=True), lg_0)
    eb = jnp.exp(lg_b - m0)
    e0 = jnp.exp(lg_0 - m0)
    den0 = jnp.sum(eb, axis=0, keepdims=True) + e0 + 1e-16
    M = _colsum_T(eb.astype(bf16), Llb)
    ws = jax.nn.sigmoid(lg_s - lg_h).astype(bf16)
    y_parts, y0_parts = [], []
    for h in range(_H1):
        sl = slice(h * _C1, (h + 1) * _C1)
        Llh, hlh = Llb[:, sl], hlb[:, sl]
        wsh = ws[:, h:h + 1]
        y_parts.append(hlh + wsh * (Llh - hlh))
        y0_parts.append((M[h:h + 1, sl] + e0[:, h:h + 1] * hl[:, sl])
                        / den0[:, h:h + 1])
    Yb = jnp.concatenate(y_parts, axis=1)
    y0 = jnp.concatenate(y0_parts, axis=1) + bias1_ref[...]
    Y = _gelu(Yb.astype(f32) + bias1_ref[...]).astype(bf16)
    y0 = _gelu(y0).astype(bf16)

    A2 = a2_ref[...]
    b2lb = b2l_ref[...].astype(bf16)
    b2rb = b2r_ref[...].astype(bf16)
    Zlb = jnp.dot(Y, w2l_ref[...], preferred_element_type=f32).astype(bf16) + b2lb
    Zrb = jnp.dot(Y, w2r_ref[...], preferred_element_type=f32).astype(bf16) + b2rb
    hl2 = jnp.dot(y0, w2l_ref[...], preferred_element_type=f32) + b2l_ref[...]
    hr2 = jnp.dot(y0, w2r_ref[...], preferred_element_type=f32) + b2r_ref[...]
    hl2b = hl2.astype(bf16)
    hr2b = hr2.astype(bf16)
    lg_s = jnp.dot(_leaky(Zlb + Zrb), A2, preferred_element_type=f32)
    lg_h = jnp.dot(_leaky(hl2b + Zrb), A2, preferred_element_type=f32)
    lg_b = jnp.dot(_leaky(Zlb + hr2b), A2, preferred_element_type=f32)
    lg_0 = jnp.dot(_leaky(hl2b + hr2b), A2, preferred_element_type=f32)
    ws = jax.nn.sigmoid(lg_s - lg_h)
    Z = hl2 + ws * (Zlb.astype(f32) - hl2) + bias2_ref[...]
    m0 = jnp.maximum(jnp.max(lg_b, axis=0, keepdims=True), lg_0)
    eb = jnp.exp(lg_b - m0)
    e0 = jnp.exp(lg_0 - m0)
    den0 = jnp.sum(eb, axis=0, keepdims=True) + e0 + 1e-16
    M2 = _colsum_T(eb.astype(bf16), Zlb)
    z0 = (M2 + e0 * hl2) / den0 + bias2_ref[...]

    mean_ref[0] = (jnp.sum(Z, axis=0, keepdims=True) + z0) * (1.0 / _N_NODE)
    g = jnp.dot(z0, wlin_ref[...], preferred_element_type=f32) + blin_ref[...]
    g_ref[0] = g + pool


def kernel(hidden_states, pooled_output, Wp1, bp1, Wp2, bp2, W1l, b1l, W1r,
           b1r, att1, bias1, W2l, b2l, W2r, b2r, att2, bias2, Wlin, blin):
    f32, bf16 = jnp.float32, jnp.bfloat16
    hs = hidden_states.reshape(_BS, _S, _DIN)
    a1 = jnp.zeros((_DH, _H1), f32)
    for h in range(_H1):
        a1 = a1.at[h * _C1:(h + 1) * _C1, h].set(att1[h])

    full2 = lambda r, c: pl.BlockSpec((r, c), lambda i: (0, 0))
    g, mean = pl.pallas_call(
        _fused_kernel,
        grid=(_BS,),
        in_specs=[
            pl.BlockSpec((1, _S, _DIN), lambda i: (i, 0, 0)),
            pl.BlockSpec((1, 1, _DH), lambda i: (i, 0, 0)),
            full2(_DIN, _DH),
            full2(1, _DH),
            full2(_DH, _DH),
            full2(1, _DH),
            full2(_DH, _DH),
            full2(1, _DH),
            full2(_DH, _DH),
            full2(1, _DH),
            full2(_DH, _H1),
            full2(1, _DH),
            full2(_DH, _DH),
            full2(1, _DH),
            full2(_DH, _DH),
            full2(1, _DH),
            full2(_DH, 1),
            full2(1, _DH),
            full2(_DH, _DH),
            full2(1, _DH),
        ],
        out_specs=[
            pl.BlockSpec((1, 1, _DH), lambda i: (i, 0, 0)),
            pl.BlockSpec((1, 1, _DH), lambda i: (i, 0, 0)),
        ],
        out_shape=[
            jax.ShapeDtypeStruct((_BS, 1, _DH), f32),
            jax.ShapeDtypeStruct((_BS, 1, _DH), f32),
        ],
        compiler_params=pltpu.CompilerParams(
            dimension_semantics=("parallel",),
            vmem_limit_bytes=64 * 1024 * 1024),
    )(hs, pooled_output.reshape(_BS, 1, _DH),
      Wp1.astype(bf16), bp1.reshape(1, _DH),
      Wp2.astype(bf16), bp2.reshape(1, _DH),
      W1l.astype(bf16), b1l.reshape(1, _DH),
      W1r.astype(bf16), b1r.reshape(1, _DH),
      a1.astype(bf16), bias1.reshape(1, _DH),
      W2l.astype(bf16), b2l.reshape(1, _DH),
      W2r.astype(bf16), b2r.reshape(1, _DH),
      att2.reshape(_DH, 1).astype(bf16), bias2.reshape(1, _DH),
      Wlin, blin.reshape(1, _DH))
    return (g.reshape(_BS, _DH), mean.reshape(_BS, _DH))


# speedup vs baseline: 97.3093x; 1.1173x over previous
import jax
import jax.numpy as jnp
from jax.experimental import pallas as pl
from jax.experimental.pallas import tpu as pltpu

_BS = 8
_S = 4096
_DIN = 768
_DH = 512
_H1, _C1 = 4, 128
_N_NODE = _S + 1

_CH = 1024
_NCH = _S // _CH


def _leaky(x):
    return jnp.maximum(x, x * jnp.asarray(0.2, x.dtype))


def _gelu(x):
    return 0.5 * x * (1.0 + jax.lax.erf(x * 0.7071067811865476))


def _colsum_T(eb, X):
    return jax.lax.dot_general(
        eb, X, (((0,), (0,)), ((), ())), preferred_element_type=jnp.float32)


def _fused_kernel(hs_ref, pool_ref, wp1_ref, bp1_ref, wp2_ref, bp2_ref,
                  w1l_ref, b1l_ref, w1r_ref, b1r_ref, a1_ref,
                  bias1_ref, w2l_ref, b2l_ref, w2r_ref, b2r_ref, a2_ref,
                  bias2_ref, wlin_ref, blin_ref, g_ref, mean_ref):
    bf16, f32 = jnp.bfloat16, jnp.float32
    pool = pool_ref[0]
    poolb = pool.astype(bf16)
    A1 = a1_ref[...]
    A2 = a2_ref[...]
    b1lb = b1l_ref[...].astype(bf16)
    b1rb = b1r_ref[...].astype(bf16)
    b2lb = b2l_ref[...].astype(bf16)
    b2rb = b2r_ref[...].astype(bf16)

    hl = jnp.dot(poolb, w1l_ref[...], preferred_element_type=f32) + b1l_ref[...]
    hr = jnp.dot(poolb, w1r_ref[...], preferred_element_type=f32) + b1r_ref[...]
    hlb = hl.astype(bf16)
    hrb = hr.astype(bf16)
    lg_0 = jnp.dot(_leaky(hlb + hrb), A1, preferred_element_type=f32)

    Llb_c, lgb_c, ws_c = [], [], []
    for c in range(_NCH):
        rows = pl.ds(c * _CH, _CH)
        h = hs_ref[0, rows, :].astype(bf16)
        t = jnp.dot(h, wp1_ref[...], preferred_element_type=f32) + bp1_ref[...]
        t = _gelu(t).astype(bf16)
        t = jnp.dot(t, wp2_ref[...], preferred_element_type=f32) + bp2_ref[...]
        P = _gelu(t).astype(bf16)
        Llb = jnp.dot(P, w1l_ref[...], preferred_element_type=f32).astype(bf16) + b1lb
        Lrb = jnp.dot(P, w1r_ref[...], preferred_element_type=f32).astype(bf16) + b1rb
        lg_s = jnp.dot(_leaky(Llb + Lrb), A1, preferred_element_type=f32)
        lg_h = jnp.dot(_leaky(hlb + Lrb), A1, preferred_element_type=f32)
        lg_b = jnp.dot(_leaky(Llb + hrb), A1, preferred_element_type=f32)
        Llb_c.append(Llb)
        lgb_c.append(lg_b)
        ws_c.append(jax.nn.sigmoid(lg_s - lg_h).astype(bf16))

    m0 = lg_0
    for c in range(_NCH):
        m0 = jnp.maximum(m0, jnp.max(lgb_c[c], axis=0, keepdims=True))
    eb_c = [jnp.exp(lgb_c[c] - m0) for c in range(_NCH)]
    e0 = jnp.exp(lg_0 - m0)
    den0 = e0 + 1e-16
    for c in range(_NCH):
        den0 = den0 + jnp.sum(eb_c[c], axis=0, keepdims=True)
    M = _colsum_T(eb_c[0].astype(bf16), Llb_c[0])
    for c in range(1, _NCH):
        M = M + _colsum_T(eb_c[c].astype(bf16), Llb_c[c])
    y0_parts = []
    for h in range(_H1):
        sl = slice(h * _C1, (h + 1) * _C1)
        y0_parts.append((M[h:h + 1, sl] + e0[:, h:h + 1] * hl[:, sl])
                        / den0[:, h:h + 1])
    y0 = jnp.concatenate(y0_parts, axis=1) + bias1_ref[...]
    y0 = _gelu(y0).astype(bf16)
    hl2 = jnp.dot(y0, w2l_ref[...], preferred_element_type=f32) + b2l_ref[...]
    hr2 = jnp.dot(y0, w2r_ref[...], preferred_element_type=f32) + b2r_ref[...]
    hl2b = hl2.astype(bf16)
    hr2b = hr2.astype(bf16)
    lg2_0 = jnp.dot(_leaky(hl2b + hr2b), A2, preferred_element_type=f32)

    Zlb_c, lgb2_c, ws2_c, zsum = [], [], [], None
    for c in range(_NCH):
        y_parts = []
        for h in range(_H1):
            sl = slice(h * _C1, (h + 1) * _C1)
            Llh, hlh = Llb_c[c][:, sl], hlb[:, sl]
            wsh = ws_c[c][:, h:h + 1]
            y_parts.append(hlh + wsh * (Llh - hlh))
        Yb = jnp.concatenate(y_parts, axis=1)
        Y = _gelu(Yb.astype(f32) + bias1_ref[...]).astype(bf16)
        Zlb = jnp.dot(Y, w2l_ref[...], preferred_element_type=f32).astype(bf16) + b2lb
        Zrb = jnp.dot(Y, w2r_ref[...], preferred_element_type=f32).astype(bf16) + b2rb
        lg_s = jnp.dot(_leaky(Zlb + Zrb), A2, preferred_element_type=f32)
        lg_h = jnp.dot(_leaky(hl2b + Zrb), A2, preferred_element_type=f32)
        lg_b = jnp.dot(_leaky(Zlb + hr2b), A2, preferred_element_type=f32)
        Zlb_c.append(Zlb)
        lgb2_c.append(lg_b)
        ws2_c.append(jax.nn.sigmoid(lg_s - lg_h))

    m0 = lg2_0
    for c in range(_NCH):
        m0 = jnp.maximum(m0, jnp.max(lgb2_c[c], axis=0, keepdims=True))
    eb2_c = [jnp.exp(lgb2_c[c] - m0) for c in range(_NCH)]
    e0 = jnp.exp(lg2_0 - m0)
    den0 = e0 + 1e-16
    for c in range(_NCH):
        den0 = den0 + jnp.sum(eb2_c[c], axis=0, keepdims=True)
    M2 = _colsum_T(eb2_c[0].astype(bf16), Zlb_c[0])
    for c in range(1, _NCH):
        M2 = M2 + _colsum_T(eb2_c[c].astype(bf16), Zlb_c[c])
    z0 = (M2 + e0 * hl2) / den0 + bias2_ref[...]

    for c in range(_NCH):
        Z = hl2 + ws2_c[c] * (Zlb_c[c].astype(f32) - hl2) + bias2_ref[...]
        s = jnp.sum(Z, axis=0, keepdims=True)
        zsum = s if zsum is None else zsum + s

    mean_ref[0] = (zsum + z0) * (1.0 / _N_NODE)
    g = jnp.dot(z0, wlin_ref[...], preferred_element_type=f32) + blin_ref[...]
    g_ref[0] = g + pool


def kernel(hidden_states, pooled_output, Wp1, bp1, Wp2, bp2, W1l, b1l, W1r,
           b1r, att1, bias1, W2l, b2l, W2r, b2r, att2, bias2, Wlin, blin):
    f32, bf16 = jnp.float32, jnp.bfloat16
    hs = hidden_states.reshape(_BS, _S, _DIN)
    a1 = jnp.zeros((_DH, _H1), f32)
    for h in range(_H1):
        a1 = a1.at[h * _C1:(h + 1) * _C1, h].set(att1[h])

    full2 = lambda r, c: pl.BlockSpec((r, c), lambda i: (0, 0))
    g, mean = pl.pallas_call(
        _fused_kernel,
        grid=(_BS,),
        in_specs=[
            pl.BlockSpec((1, _S, _DIN), lambda i: (i, 0, 0)),
            pl.BlockSpec((1, 1, _DH), lambda i: (i, 0, 0)),
            full2(_DIN, _DH),
            full2(1, _DH),
            full2(_DH, _DH),
            full2(1, _DH),
            full2(_DH, _DH),
            full2(1, _DH),
            full2(_DH, _DH),
            full2(1, _DH),
            full2(_DH, _H1),
            full2(1, _DH),
            full2(_DH, _DH),
            full2(1, _DH),
            full2(_DH, _DH),
            full2(1, _DH),
            full2(_DH, 1),
            full2(1, _DH),
            full2(_DH, _DH),
            full2(1, _DH),
        ],
        out_specs=[
            pl.BlockSpec((1, 1, _DH), lambda i: (i, 0, 0)),
            pl.BlockSpec((1, 1, _DH), lambda i: (i, 0, 0)),
        ],
        out_shape=[
            jax.ShapeDtypeStruct((_BS, 1, _DH), f32),
            jax.ShapeDtypeStruct((_BS, 1, _DH), f32),
        ],
        compiler_params=pltpu.CompilerParams(
            dimension_semantics=("parallel",),
            vmem_limit_bytes=64 * 1024 * 1024),
    )(hs, pooled_output.reshape(_BS, 1, _DH),
      Wp1.astype(bf16), bp1.reshape(1, _DH),
      Wp2.astype(bf16), bp2.reshape(1, _DH),
      W1l.astype(bf16), b1l.reshape(1, _DH),
      W1r.astype(bf16), b1r.reshape(1, _DH),
      a1.astype(bf16), bias1.reshape(1, _DH),
      W2l.astype(bf16), b2l.reshape(1, _DH),
      W2r.astype(bf16), b2r.reshape(1, _DH),
      att2.reshape(_DH, 1).astype(bf16), bias2.reshape(1, _DH),
      Wlin, blin.reshape(1, _DH))
    return (g.reshape(_BS, _DH), mean.reshape(_BS, _DH))


# speedup vs baseline: 98.0918x; 1.0080x over previous
import jax
import jax.numpy as jnp
from jax.experimental import pallas as pl
from jax.experimental.pallas import tpu as pltpu

_BS = 8
_S = 4096
_DIN = 768
_DH = 512
_H1, _C1 = 4, 128
_N_NODE = _S + 1

_CH = 512
_NCH = _S // _CH


def _leaky(x):
    return jnp.maximum(x, x * jnp.asarray(0.2, x.dtype))


def _gelu(x):
    return 0.5 * x * (1.0 + jax.lax.erf(x * 0.7071067811865476))


def _colsum_T(eb, X):
    return jax.lax.dot_general(
        eb, X, (((0,), (0,)), ((), ())), preferred_element_type=jnp.float32)


def _fused_kernel(hs_ref, pool_ref, wp1_ref, bp1_ref, wp2_ref, bp2_ref,
                  w1l_ref, b1l_ref, w1r_ref, b1r_ref, a1_ref,
                  bias1_ref, w2l_ref, b2l_ref, w2r_ref, b2r_ref, a2_ref,
                  bias2_ref, wlin_ref, blin_ref, g_ref, mean_ref):
    bf16, f32 = jnp.bfloat16, jnp.float32
    pool = pool_ref[0]
    poolb = pool.astype(bf16)
    A1 = a1_ref[...]
    A2 = a2_ref[...]
    b1lb = b1l_ref[...].astype(bf16)
    b1rb = b1r_ref[...].astype(bf16)
    b2lb = b2l_ref[...].astype(bf16)
    b2rb = b2r_ref[...].astype(bf16)

    hl = jnp.dot(poolb, w1l_ref[...], preferred_element_type=f32) + b1l_ref[...]
    hr = jnp.dot(poolb, w1r_ref[...], preferred_element_type=f32) + b1r_ref[...]
    hlb = hl.astype(bf16)
    hrb = hr.astype(bf16)
    lg_0 = jnp.dot(_leaky(hlb + hrb), A1, preferred_element_type=f32)

    Llb_c, lgb_c, ws_c = [], [], []
    for c in range(_NCH):
        rows = pl.ds(c * _CH, _CH)
        h = hs_ref[0, rows, :].astype(bf16)
        t = jnp.dot(h, wp1_ref[...], preferred_element_type=f32) + bp1_ref[...]
        t = _gelu(t).astype(bf16)
        t = jnp.dot(t, wp2_ref[...], preferred_element_type=f32) + bp2_ref[...]
        P = _gelu(t).astype(bf16)
        Llb = jnp.dot(P, w1l_ref[...], preferred_element_type=f32).astype(bf16) + b1lb
        Lrb = jnp.dot(P, w1r_ref[...], preferred_element_type=f32).astype(bf16) + b1rb
        lg_s = jnp.dot(_leaky(Llb + Lrb), A1, preferred_element_type=f32)
        lg_h = jnp.dot(_leaky(hlb + Lrb), A1, preferred_element_type=f32)
        lg_b = jnp.dot(_leaky(Llb + hrb), A1, preferred_element_type=f32)
        Llb_c.append(Llb)
        lgb_c.append(lg_b)
        ws_c.append(jax.nn.sigmoid(lg_s - lg_h).astype(bf16))

    m0 = lg_0
    for c in range(_NCH):
        m0 = jnp.maximum(m0, jnp.max(lgb_c[c], axis=0, keepdims=True))
    eb_c = [jnp.exp(lgb_c[c] - m0) for c in range(_NCH)]
    e0 = jnp.exp(lg_0 - m0)
    den0 = e0 + 1e-16
    for c in range(_NCH):
        den0 = den0 + jnp.sum(eb_c[c], axis=0, keepdims=True)
    M = _colsum_T(eb_c[0].astype(bf16), Llb_c[0])
    for c in range(1, _NCH):
        M = M + _colsum_T(eb_c[c].astype(bf16), Llb_c[c])
    y0_parts = []
    for h in range(_H1):
        sl = slice(h * _C1, (h + 1) * _C1)
        y0_parts.append((M[h:h + 1, sl] + e0[:, h:h + 1] * hl[:, sl])
                        / den0[:, h:h + 1])
    y0 = jnp.concatenate(y0_parts, axis=1) + bias1_ref[...]
    y0 = _gelu(y0).astype(bf16)
    hl2 = jnp.dot(y0, w2l_ref[...], preferred_element_type=f32) + b2l_ref[...]
    hr2 = jnp.dot(y0, w2r_ref[...], preferred_element_type=f32) + b2r_ref[...]
    hl2b = hl2.astype(bf16)
    hr2b = hr2.astype(bf16)
    lg2_0 = jnp.dot(_leaky(hl2b + hr2b), A2, preferred_element_type=f32)

    Zlb_c, lgb2_c, ws2_c, zsum = [], [], [], None
    for c in range(_NCH):
        y_parts = []
        for h in range(_H1):
            sl = slice(h * _C1, (h + 1) * _C1)
            Llh, hlh = Llb_c[c][:, sl], hlb[:, sl]
            wsh = ws_c[c][:, h:h + 1]
            y_parts.append(hlh + wsh * (Llh - hlh))
        Yb = jnp.concatenate(y_parts, axis=1)
        Y = _gelu(Yb.astype(f32) + bias1_ref[...]).astype(bf16)
        Zlb = jnp.dot(Y, w2l_ref[...], preferred_element_type=f32).astype(bf16) + b2lb
        Zrb = jnp.dot(Y, w2r_ref[...], preferred_element_type=f32).astype(bf16) + b2rb
        lg_s = jnp.dot(_leaky(Zlb + Zrb), A2, preferred_element_type=f32)
        lg_h = jnp.dot(_leaky(hl2b + Zrb), A2, preferred_element_type=f32)
        lg_b = jnp.dot(_leaky(Zlb + hr2b), A2, preferred_element_type=f32)
        Zlb_c.append(Zlb)
        lgb2_c.append(lg_b)
        ws2_c.append(jax.nn.sigmoid(lg_s - lg_h))

    m0 = lg2_0
    for c in range(_NCH):
        m0 = jnp.maximum(m0, jnp.max(lgb2_c[c], axis=0, keepdims=True))
    eb2_c = [jnp.exp(lgb2_c[c] - m0) for c in range(_NCH)]
    e0 = jnp.exp(lg2_0 - m0)
    den0 = e0 + 1e-16
    for c in range(_NCH):
        den0 = den0 + jnp.sum(eb2_c[c], axis=0, keepdims=True)
    M2 = _colsum_T(eb2_c[0].astype(bf16), Zlb_c[0])
    for c in range(1, _NCH):
        M2 = M2 + _colsum_T(eb2_c[c].astype(bf16), Zlb_c[c])
    z0 = (M2 + e0 * hl2) / den0 + bias2_ref[...]

    for c in range(_NCH):
        Z = hl2 + ws2_c[c] * (Zlb_c[c].astype(f32) - hl2) + bias2_ref[...]
        s = jnp.sum(Z, axis=0, keepdims=True)
        zsum = s if zsum is None else zsum + s

    mean_ref[0] = (zsum + z0) * (1.0 / _N_NODE)
    g = jnp.dot(z0, wlin_ref[...], preferred_element_type=f32) + blin_ref[...]
    g_ref[0] = g + pool


def kernel(hidden_states, pooled_output, Wp1, bp1, Wp2, bp2, W1l, b1l, W1r,
           b1r, att1, bias1, W2l, b2l, W2r, b2r, att2, bias2, Wlin, blin):
    f32, bf16 = jnp.float32, jnp.bfloat16
    hs = hidden_states.reshape(_BS, _S, _DIN)
    a1 = jnp.zeros((_DH, _H1), f32)
    for h in range(_H1):
        a1 = a1.at[h * _C1:(h + 1) * _C1, h].set(att1[h])

    full2 = lambda r, c: pl.BlockSpec((r, c), lambda i: (0, 0))
    g, mean = pl.pallas_call(
        _fused_kernel,
        grid=(_BS,),
        in_specs=[
            pl.BlockSpec((1, _S, _DIN), lambda i: (i, 0, 0)),
            pl.BlockSpec((1, 1, _DH), lambda i: (i, 0, 0)),
            full2(_DIN, _DH),
            full2(1, _DH),
            full2(_DH, _DH),
            full2(1, _DH),
            full2(_DH, _DH),
            full2(1, _DH),
            full2(_DH, _DH),
            full2(1, _DH),
            full2(_DH, _H1),
            full2(1, _DH),
            full2(_DH, _DH),
            full2(1, _DH),
            full2(_DH, _DH),
            full2(1, _DH),
            full2(_DH, 1),
            full2(1, _DH),
            full2(_DH, _DH),
            full2(1, _DH),
        ],
        out_specs=[
            pl.BlockSpec((1, 1, _DH), lambda i: (i, 0, 0)),
            pl.BlockSpec((1, 1, _DH), lambda i: (i, 0, 0)),
        ],
        out_shape=[
            jax.ShapeDtypeStruct((_BS, 1, _DH), f32),
            jax.ShapeDtypeStruct((_BS, 1, _DH), f32),
        ],
        compiler_params=pltpu.CompilerParams(
            dimension_semantics=("parallel",),
            vmem_limit_bytes=64 * 1024 * 1024),
    )(hs, pooled_output.reshape(_BS, 1, _DH),
      Wp1.astype(bf16), bp1.reshape(1, _DH),
      Wp2.astype(bf16), bp2.reshape(1, _DH),
      W1l.astype(bf16), b1l.reshape(1, _DH),
      W1r.astype(bf16), b1r.reshape(1, _DH),
      a1.astype(bf16), bias1.reshape(1, _DH),
      W2l.astype(bf16), b2l.reshape(1, _DH),
      W2r.astype(bf16), b2r.reshape(1, _DH),
      att2.reshape(_DH, 1).astype(bf16), bias2.reshape(1, _DH),
      Wlin, blin.reshape(1, _DH))
    return (g.reshape(_BS, _DH), mean.reshape(_BS, _DH))


# speedup vs baseline: 106.5697x; 1.0864x over previous
import jax
import jax.numpy as jnp
from jax.experimental import pallas as pl
from jax.experimental.pallas import tpu as pltpu

_BS = 8
_S = 4096
_DIN = 768
_DH = 512
_H1, _C1 = 4, 128
_N_NODE = _S + 1

_CH = 512
_NCH = _S // _CH


def _leaky(x):
    return jnp.maximum(x, x * jnp.asarray(0.2, x.dtype))


def _gelu(x):
    return 0.5 * x * (1.0 + jax.lax.erf(x * 0.7071067811865476))


def _prep_kernel(wp1_ref, wp2_ref, w1l_ref, w1r_ref, w2l_ref, w2r_ref,
                 att1_ref, att2_ref,
                 wp1b_ref, wp2b_ref, w1lb_ref, w1rb_ref, w2lb_ref, w2rb_ref,
                 a1_ref, a2_ref):
    bf16 = jnp.bfloat16
    wp1b_ref[...] = wp1_ref[...].astype(bf16)
    wp2b_ref[...] = wp2_ref[...].astype(bf16)
    w1lb_ref[...] = w1l_ref[...].astype(bf16)
    w1rb_ref[...] = w1r_ref[...].astype(bf16)
    w2lb_ref[...] = w2l_ref[...].astype(bf16)
    w2rb_ref[...] = w2r_ref[...].astype(bf16)
    col1 = jnp.swapaxes(att1_ref[...], 0, 1)
    row_head = jax.lax.broadcasted_iota(jnp.int32, (_DH, _H1), 0) // _C1
    col_head = jax.lax.broadcasted_iota(jnp.int32, (_DH, _H1), 1)
    a1_ref[...] = jnp.where(row_head == col_head, col1, 0.0).astype(bf16)
    a2_ref[...] = jnp.swapaxes(att2_ref[...], 0, 1).astype(bf16)


def _colsum_T(eb, X):
    return jax.lax.dot_general(
        eb, X, (((0,), (0,)), ((), ())), preferred_element_type=jnp.float32)


def _fused_kernel(hs_ref, pool_ref, wp1_ref, bp1_ref, wp2_ref, bp2_ref,
                  w1l_ref, b1l_ref, w1r_ref, b1r_ref, a1_ref,
                  bias1_ref, w2l_ref, b2l_ref, w2r_ref, b2r_ref, a2_ref,
                  bias2_ref, wlin_ref, blin_ref, g_ref, mean_ref):
    bf16, f32 = jnp.bfloat16, jnp.float32
    pool = pool_ref[0]
    poolb = pool.astype(bf16)
    A1 = a1_ref[...]
    A2 = a2_ref[...]
    b1lb = b1l_ref[...].astype(bf16)
    b1rb = b1r_ref[...].astype(bf16)
    b2lb = b2l_ref[...].astype(bf16)
    b2rb = b2r_ref[...].astype(bf16)

    hl = jnp.dot(poolb, w1l_ref[...], preferred_element_type=f32) + b1l_ref[...]
    hr = jnp.dot(poolb, w1r_ref[...], preferred_element_type=f32) + b1r_ref[...]
    hlb = hl.astype(bf16)
    hrb = hr.astype(bf16)
    lg_0 = jnp.dot(_leaky(hlb + hrb), A1, preferred_element_type=f32)

    Llb_c, lgb_c, ws_c = [], [], []
    for c in range(_NCH):
        rows = pl.ds(c * _CH, _CH)
        h = hs_ref[0, rows, :].astype(bf16)
        t = jnp.dot(h, wp1_ref[...], preferred_element_type=f32) + bp1_ref[...]
        t = _gelu(t).astype(bf16)
        t = jnp.dot(t, wp2_ref[...], preferred_element_type=f32) + bp2_ref[...]
        P = _gelu(t).astype(bf16)
        Llb = jnp.dot(P, w1l_ref[...], preferred_element_type=f32).astype(bf16) + b1lb
        Lrb = jnp.dot(P, w1r_ref[...], preferred_element_type=f32).astype(bf16) + b1rb
        lg_s = jnp.dot(_leaky(Llb + Lrb), A1, preferred_element_type=f32)
        lg_h = jnp.dot(_leaky(hlb + Lrb), A1, preferred_element_type=f32)
        lg_b = jnp.dot(_leaky(Llb + hrb), A1, preferred_element_type=f32)
        Llb_c.append(Llb)
        lgb_c.append(lg_b)
        ws_c.append(jax.nn.sigmoid(lg_s - lg_h).astype(bf16))

    m0 = lg_0
    for c in range(_NCH):
        m0 = jnp.maximum(m0, jnp.max(lgb_c[c], axis=0, keepdims=True))
    eb_c = [jnp.exp(lgb_c[c] - m0) for c in range(_NCH)]
    e0 = jnp.exp(lg_0 - m0)
    den0 = e0 + 1e-16
    for c in range(_NCH):
        den0 = den0 + jnp.sum(eb_c[c], axis=0, keepdims=True)
    M = _colsum_T(eb_c[0].astype(bf16), Llb_c[0])
    for c in range(1, _NCH):
        M = M + _colsum_T(eb_c[c].astype(bf16), Llb_c[c])
    y0_parts = []
    for h in range(_H1):
        sl = slice(h * _C1, (h + 1) * _C1)
        y0_parts.append((M[h:h + 1, sl] + e0[:, h:h + 1] * hl[:, sl])
                        / den0[:, h:h + 1])
    y0 = jnp.concatenate(y0_parts, axis=1) + bias1_ref[...]
    y0 = _gelu(y0).astype(bf16)
    hl2 = jnp.dot(y0, w2l_ref[...], preferred_element_type=f32) + b2l_ref[...]
    hr2 = jnp.dot(y0, w2r_ref[...], preferred_element_type=f32) + b2r_ref[...]
    hl2b = hl2.astype(bf16)
    hr2b = hr2.astype(bf16)
    lg2_0 = jnp.dot(_leaky(hl2b + hr2b), A2, preferred_element_type=f32)

    Zlb_c, lgb2_c, ws2_c, zsum = [], [], [], None
    for c in range(_NCH):
        y_parts = []
        for h in range(_H1):
            sl = slice(h * _C1, (h + 1) * _C1)
            Llh, hlh = Llb_c[c][:, sl], hlb[:, sl]
            wsh = ws_c[c][:, h:h + 1]
            y_parts.append(hlh + wsh * (Llh - hlh))
        Yb = jnp.concatenate(y_parts, axis=1)
        Y = _gelu(Yb.astype(f32) + bias1_ref[...]).astype(bf16)
        Zlb = jnp.dot(Y, w2l_ref[...], preferred_element_type=f32).astype(bf16) + b2lb
        Zrb = jnp.dot(Y, w2r_ref[...], preferred_element_type=f32).astype(bf16) + b2rb
        lg_s = jnp.dot(_leaky(Zlb + Zrb), A2, preferred_element_type=f32)
        lg_h = jnp.dot(_leaky(hl2b + Zrb), A2, preferred_element_type=f32)
        lg_b = jnp.dot(_leaky(Zlb + hr2b), A2, preferred_element_type=f32)
        Zlb_c.append(Zlb)
        lgb2_c.append(lg_b)
        ws2_c.append(jax.nn.sigmoid(lg_s - lg_h))

    m0 = lg2_0
    for c in range(_NCH):
        m0 = jnp.maximum(m0, jnp.max(lgb2_c[c], axis=0, keepdims=True))
    eb2_c = [jnp.exp(lgb2_c[c] - m0) for c in range(_NCH)]
    e0 = jnp.exp(lg2_0 - m0)
    den0 = e0 + 1e-16
    for c in range(_NCH):
        den0 = den0 + jnp.sum(eb2_c[c], axis=0, keepdims=True)
    M2 = _colsum_T(eb2_c[0].astype(bf16), Zlb_c[0])
    for c in range(1, _NCH):
        M2 = M2 + _colsum_T(eb2_c[c].astype(bf16), Zlb_c[c])
    z0 = (M2 + e0 * hl2) / den0 + bias2_ref[...]

    for c in range(_NCH):
        Z = hl2 + ws2_c[c] * (Zlb_c[c].astype(f32) - hl2) + bias2_ref[...]
        s = jnp.sum(Z, axis=0, keepdims=True)
        zsum = s if zsum is None else zsum + s

    mean_ref[0] = (zsum + z0) * (1.0 / _N_NODE)
    g = jnp.dot(z0, wlin_ref[...], preferred_element_type=f32) + blin_ref[...]
    g_ref[0] = g + pool


def kernel(hidden_states, pooled_output, Wp1, bp1, Wp2, bp2, W1l, b1l, W1r,
           b1r, att1, bias1, W2l, b2l, W2r, b2r, att2, bias2, Wlin, blin):
    f32, bf16 = jnp.float32, jnp.bfloat16
    hs = hidden_states.reshape(_BS, _S, _DIN)

    wp1b, wp2b, w1lb, w1rb, w2lb, w2rb, a1, a2 = pl.pallas_call(
        _prep_kernel,
        out_shape=[
            jax.ShapeDtypeStruct((_DIN, _DH), bf16),
            jax.ShapeDtypeStruct((_DH, _DH), bf16),
            jax.ShapeDtypeStruct((_DH, _DH), bf16),
            jax.ShapeDtypeStruct((_DH, _DH), bf16),
            jax.ShapeDtypeStruct((_DH, _DH), bf16),
            jax.ShapeDtypeStruct((_DH, _DH), bf16),
            jax.ShapeDtypeStruct((_DH, _H1), bf16),
            jax.ShapeDtypeStruct((_DH, 1), bf16),
        ],
    )(Wp1, Wp2, W1l, W1r, W2l, W2r, att1.reshape(1, _DH), att2)

    full2 = lambda r, c: pl.BlockSpec((r, c), lambda i: (0, 0))
    g, mean = pl.pallas_call(
        _fused_kernel,
        grid=(_BS,),
        in_specs=[
            pl.BlockSpec((1, _S, _DIN), lambda i: (i, 0, 0)),
            pl.BlockSpec((1, 1, _DH), lambda i: (i, 0, 0)),
            full2(_DIN, _DH),
            full2(1, _DH),
            full2(_DH, _DH),
            full2(1, _DH),
            full2(_DH, _DH),
            full2(1, _DH),
            full2(_DH, _DH),
            full2(1, _DH),
            full2(_DH, _H1),
            full2(1, _DH),
            full2(_DH, _DH),
            full2(1, _DH),
            full2(_DH, _DH),
            full2(1, _DH),
            full2(_DH, 1),
            full2(1, _DH),
            full2(_DH, _DH),
            full2(1, _DH),
        ],
        out_specs=[
            pl.BlockSpec((1, 1, _DH), lambda i: (i, 0, 0)),
            pl.BlockSpec((1, 1, _DH), lambda i: (i, 0, 0)),
        ],
        out_shape=[
            jax.ShapeDtypeStruct((_BS, 1, _DH), f32),
            jax.ShapeDtypeStruct((_BS, 1, _DH), f32),
        ],
        compiler_params=pltpu.CompilerParams(
            dimension_semantics=("parallel",),
            vmem_limit_bytes=64 * 1024 * 1024),
    )(hs, pooled_output.reshape(_BS, 1, _DH),
      wp1b, bp1.reshape(1, _DH),
      wp2b, bp2.reshape(1, _DH),
      w1lb, b1l.reshape(1, _DH),
      w1rb, b1r.reshape(1, _DH),
      a1, bias1.reshape(1, _DH),
      w2lb, b2l.reshape(1, _DH),
      w2rb, b2r.reshape(1, _DH),
      a2, bias2.reshape(1, _DH),
      Wlin, blin.reshape(1, _DH))
    return (g.reshape(_BS, _DH), mean.reshape(_BS, _DH))


# speedup vs baseline: 107.2447x; 1.0063x over previous
import jax
import jax.numpy as jnp
from jax.experimental import pallas as pl
from jax.experimental.pallas import tpu as pltpu

_BS = 8
_S = 4096
_DIN = 768
_DH = 512
_H1, _C1 = 4, 128
_N_NODE = _S + 1

_CH = 512
_NCH = _S // _CH


def _leaky(x):
    return jnp.maximum(x, x * jnp.asarray(0.2, x.dtype))


def _gelu(x):
    return 0.5 * x * (1.0 + jax.lax.erf(x * 0.7071067811865476))


def _prep_kernel(wp1_ref, wp2_ref, w1l_ref, w1r_ref, w2l_ref, w2r_ref,
                 att1_ref, att2_ref,
                 wp1b_ref, wp2b_ref, w1lb_ref, w1rb_ref, w2lb_ref, w2rb_ref,
                 a1_ref, a2_ref):
    bf16 = jnp.bfloat16
    wp1b_ref[...] = wp1_ref[...].astype(bf16)
    wp2b_ref[...] = wp2_ref[...].astype(bf16)
    w1lb_ref[...] = w1l_ref[...].astype(bf16)
    w1rb_ref[...] = w1r_ref[...].astype(bf16)
    w2lb_ref[...] = w2l_ref[...].astype(bf16)
    w2rb_ref[...] = w2r_ref[...].astype(bf16)
    col1 = jnp.swapaxes(att1_ref[...], 0, 1)
    row_head = jax.lax.broadcasted_iota(jnp.int32, (_DH, _H1), 0) // _C1
    col_head = jax.lax.broadcasted_iota(jnp.int32, (_DH, _H1), 1)
    a1_ref[...] = jnp.where(row_head == col_head, col1, 0.0).astype(bf16)
    a2_ref[...] = jnp.swapaxes(att2_ref[...], 0, 1).astype(bf16)


def _colsum_T(eb, X):
    return jax.lax.dot_general(
        eb, X, (((0,), (0,)), ((), ())), preferred_element_type=jnp.float32)


def _fused_kernel(hs_ref, pool_ref, wp1_ref, bp1_ref, wp2_ref, bp2_ref,
                  w1l_ref, b1l_ref, w1r_ref, b1r_ref, a1_ref,
                  bias1_ref, w2l_ref, b2l_ref, w2r_ref, b2r_ref, a2_ref,
                  bias2_ref, wlin_ref, blin_ref, g_ref, mean_ref):
    bf16, f32 = jnp.bfloat16, jnp.float32
    pool = pool_ref[0]
    poolb = pool.astype(bf16)
    A1 = a1_ref[...]
    A2 = a2_ref[...]
    b1lb = b1l_ref[...].astype(bf16)
    b1rb = b1r_ref[...].astype(bf16)
    b2lb = b2l_ref[...].astype(bf16)
    b2rb = b2r_ref[...].astype(bf16)

    hl = jnp.dot(poolb, w1l_ref[...], preferred_element_type=f32) + b1l_ref[...]
    hr = jnp.dot(poolb, w1r_ref[...], preferred_element_type=f32) + b1r_ref[...]
    hlb = hl.astype(bf16)
    hrb = hr.astype(bf16)
    lg_0 = jnp.dot(_leaky(hlb + hrb), A1, preferred_element_type=f32)

    Llb_c, lgb_c, ws_c = [], [], []
    for c in range(_NCH):
        rows = pl.ds(c * _CH, _CH)
        h = hs_ref[0, rows, :].astype(bf16)
        t = jnp.dot(h, wp1_ref[...], preferred_element_type=f32) + bp1_ref[...]
        t = _gelu(t).astype(bf16)
        t = jnp.dot(t, wp2_ref[...], preferred_element_type=f32) + bp2_ref[...]
        P = _gelu(t).astype(bf16)
        Llb = jnp.dot(P, w1l_ref[...], preferred_element_type=f32).astype(bf16) + b1lb
        Lrb = jnp.dot(P, w1r_ref[...], preferred_element_type=f32).astype(bf16) + b1rb
        lg_s = jnp.dot(_leaky(Llb + Lrb), A1, preferred_element_type=f32)
        lg_h = jnp.dot(_leaky(hlb + Lrb), A1, preferred_element_type=f32)
        lg_b = jnp.dot(_leaky(Llb + hrb), A1, preferred_element_type=f32)
        Llb_c.append(Llb)
        lgb_c.append(lg_b)
        ws_c.append(jax.nn.sigmoid(lg_s - lg_h).astype(bf16))

    m0 = lg_0
    for c in range(_NCH):
        m0 = jnp.maximum(m0, jnp.max(lgb_c[c], axis=0, keepdims=True))
    eb_c = [jnp.exp(lgb_c[c] - m0) for c in range(_NCH)]
    e0 = jnp.exp(lg_0 - m0)
    den0 = e0 + 1e-16
    for c in range(_NCH):
        den0 = den0 + jnp.sum(eb_c[c], axis=0, keepdims=True)
    M = _colsum_T(eb_c[0].astype(bf16), Llb_c[0])
    for c in range(1, _NCH):
        M = M + _colsum_T(eb_c[c].astype(bf16), Llb_c[c])
    y0_parts = []
    for h in range(_H1):
        sl = slice(h * _C1, (h + 1) * _C1)
        y0_parts.append((M[h:h + 1, sl] + e0[:, h:h + 1] * hl[:, sl])
                        / den0[:, h:h + 1])
    y0 = jnp.concatenate(y0_parts, axis=1) + bias1_ref[...]
    y0 = _gelu(y0).astype(bf16)
    hl2 = jnp.dot(y0, w2l_ref[...], preferred_element_type=f32) + b2l_ref[...]
    hr2 = jnp.dot(y0, w2r_ref[...], preferred_element_type=f32) + b2r_ref[...]
    hl2b = hl2.astype(bf16)
    hr2b = hr2.astype(bf16)
    lg2_0 = jnp.dot(_leaky(hl2b + hr2b), A2, preferred_element_type=f32)

    Zlb_c, lgb2_c, zsum = [], [], None
    for c in range(_NCH):
        y_parts = []
        for h in range(_H1):
            sl = slice(h * _C1, (h + 1) * _C1)
            Llh, hlh = Llb_c[c][:, sl], hlb[:, sl]
            wsh = ws_c[c][:, h:h + 1]
            y_parts.append(hlh + wsh * (Llh - hlh))
        Yb = jnp.concatenate(y_parts, axis=1)
        Y = _gelu(Yb.astype(f32) + bias1_ref[...]).astype(bf16)
        Zlb = jnp.dot(Y, w2l_ref[...], preferred_element_type=f32).astype(bf16) + b2lb
        Zrb = jnp.dot(Y, w2r_ref[...], preferred_element_type=f32).astype(bf16) + b2rb
        lg_s = jnp.dot(_leaky(Zlb + Zrb), A2, preferred_element_type=f32)
        lg_h = jnp.dot(_leaky(hl2b + Zrb), A2, preferred_element_type=f32)
        lg_b = jnp.dot(_leaky(Zlb + hr2b), A2, preferred_element_type=f32)
        Zlb_c.append(Zlb)
        lgb2_c.append(lg_b)
        ws2 = jax.nn.sigmoid(lg_s - lg_h)
        Z = hl2 + ws2 * (Zlb.astype(f32) - hl2) + bias2_ref[...]
        s = jnp.sum(Z, axis=0, keepdims=True)
        zsum = s if zsum is None else zsum + s

    m0 = lg2_0
    for c in range(_NCH):
        m0 = jnp.maximum(m0, jnp.max(lgb2_c[c], axis=0, keepdims=True))
    eb2_c = [jnp.exp(lgb2_c[c] - m0) for c in range(_NCH)]
    e0 = jnp.exp(lg2_0 - m0)
    den0 = e0 + 1e-16
    for c in range(_NCH):
        den0 = den0 + jnp.sum(eb2_c[c], axis=0, keepdims=True)
    M2 = _colsum_T(eb2_c[0].astype(bf16), Zlb_c[0])
    for c in range(1, _NCH):
        M2 = M2 + _colsum_T(eb2_c[c].astype(bf16), Zlb_c[c])
    z0 = (M2 + e0 * hl2) / den0 + bias2_ref[...]

    mean_ref[0] = (zsum + z0) * (1.0 / _N_NODE)
    g = jnp.dot(z0, wlin_ref[...], preferred_element_type=f32) + blin_ref[...]
    g_ref[0] = g + pool


def kernel(hidden_states, pooled_output, Wp1, bp1, Wp2, bp2, W1l, b1l, W1r,
           b1r, att1, bias1, W2l, b2l, W2r, b2r, att2, bias2, Wlin, blin):
    f32, bf16 = jnp.float32, jnp.bfloat16
    hs = hidden_states.reshape(_BS, _S, _DIN)

    wp1b, wp2b, w1lb, w1rb, w2lb, w2rb, a1, a2 = pl.pallas_call(
        _prep_kernel,
        out_shape=[
            jax.ShapeDtypeStruct((_DIN, _DH), bf16),
            jax.ShapeDtypeStruct((_DH, _DH), bf16),
            jax.ShapeDtypeStruct((_DH, _DH), bf16),
            jax.ShapeDtypeStruct((_DH, _DH), bf16),
            jax.ShapeDtypeStruct((_DH, _DH), bf16),
            jax.ShapeDtypeStruct((_DH, _DH), bf16),
            jax.ShapeDtypeStruct((_DH, _H1), bf16),
            jax.ShapeDtypeStruct((_DH, 1), bf16),
        ],
    )(Wp1, Wp2, W1l, W1r, W2l, W2r, att1.reshape(1, _DH), att2)

    full2 = lambda r, c: pl.BlockSpec((r, c), lambda i: (0, 0))
    g, mean = pl.pallas_call(
        _fused_kernel,
        grid=(_BS,),
        in_specs=[
            pl.BlockSpec((1, _S, _DIN), lambda i: (i, 0, 0)),
            pl.BlockSpec((1, 1, _DH), lambda i: (i, 0, 0)),
            full2(_DIN, _DH),
            full2(1, _DH),
            full2(_DH, _DH),
            full2(1, _DH),
            full2(_DH, _DH),
            full2(1, _DH),
            full2(_DH, _DH),
            full2(1, _DH),
            full2(_DH, _H1),
            full2(1, _DH),
            full2(_DH, _DH),
            full2(1, _DH),
            full2(_DH, _DH),
            full2(1, _DH),
            full2(_DH, 1),
            full2(1, _DH),
            full2(_DH, _DH),
            full2(1, _DH),
        ],
        out_specs=[
            pl.BlockSpec((1, 1, _DH), lambda i: (i, 0, 0)),
            pl.BlockSpec((1, 1, _DH), lambda i: (i, 0, 0)),
        ],
        out_shape=[
            jax.ShapeDtypeStruct((_BS, 1, _DH), f32),
            jax.ShapeDtypeStruct((_BS, 1, _DH), f32),
        ],
        compiler_params=pltpu.CompilerParams(
            dimension_semantics=("parallel",),
            vmem_limit_bytes=64 * 1024 * 1024),
    )(hs, pooled_output.reshape(_BS, 1, _DH),
      wp1b, bp1.reshape(1, _DH),
      wp2b, bp2.reshape(1, _DH),
      w1lb, b1l.reshape(1, _DH),
      w1rb, b1r.reshape(1, _DH),
      a1, bias1.reshape(1, _DH),
      w2lb, b2l.reshape(1, _DH),
      w2rb, b2r.reshape(1, _DH),
      a2, bias2.reshape(1, _DH),
      Wlin, blin.reshape(1, _DH))
    return (g.reshape(_BS, _DH), mean.reshape(_BS, _DH))
